# baseline (speedup 1.0000x reference)
import functools

import jax
import jax.numpy as jnp
from jax import lax
from jax.experimental import pallas as pl
from jax.experimental.pallas import tpu as pltpu
from jax.experimental.pallas import tpu_sc as plsc

EPS = 1e-07
N_DATA = 100000
FEAT = 128
BSZ = 1024
K = 511
P = 1
KP = K + P
T_NCE = 0.07

NC = 2
NS = 16
NW = NC * NS
B_PER_W = BSZ // NW
CHUNK = 128
NCHUNK = KP // CHUNK
NBUF = 2


def _emb_body(fs_ref, ws_ref, bs_ref, ft_ref, wt_ref, bt_ref, out_ref):
    for side, (f, w, b) in enumerate(
        ((fs_ref, ws_ref, bs_ref), (ft_ref, wt_ref, bt_ref))
    ):
        e = jnp.dot(
            f[...], w[...],
            preferred_element_type=jnp.float32,
            precision=lax.Precision.HIGHEST,
        ) + b[...]
        e = e / jnp.sqrt(jnp.sum(e * e, axis=1, keepdims=True))
        out_ref[side] = e


def _embed(f_s, W_s, b_s, f_t, W_t, b_t):
    blk = 256
    grid = (BSZ // blk,)
    return pl.pallas_call(
        _emb_body,
        grid=grid,
        in_specs=[
            pl.BlockSpec((blk, f_s.shape[1]), lambda i: (i, 0)),
            pl.BlockSpec(W_s.shape, lambda i: (0, 0)),
            pl.BlockSpec((1, FEAT), lambda i: (0, 0)),
            pl.BlockSpec((blk, f_t.shape[1]), lambda i: (i, 0)),
            pl.BlockSpec(W_t.shape, lambda i: (0, 0)),
            pl.BlockSpec((1, FEAT), lambda i: (0, 0)),
        ],
        out_specs=pl.BlockSpec((2, blk, FEAT), lambda i: (0, i, 0)),
        out_shape=jax.ShapeDtypeStruct((2, BSZ, FEAT), jnp.float32),
    )(f_s, W_s, b_s.reshape(1, FEAT), f_t, W_t, b_t.reshape(1, FEAT))


def _sc_body(mem_t, mem_s, inds, emb, out, idx_all, emb_all, rows_v, out_all,
             sem0, sem1, sem2, sem3):
    wid = lax.axis_index("s") * NC + lax.axis_index("c")
    base = wid * B_PER_W
    lane = lax.iota(jnp.int32, 16)
    sems = (sem0, sem1, sem2, sem3)
    NT = B_PER_W * NCHUNK

    pltpu.sync_copy(inds.at[pl.ds(base, B_PER_W)], idx_all)

    for side in range(2):
        bank = mem_t if side == 0 else mem_s
        pltpu.sync_copy(emb.at[side, pl.ds(base, B_PER_W)], emb_all)

        for p0 in range(NBUF - 1):
            pltpu.make_async_copy(
                bank.at[idx_all.at[p0 // NCHUNK, p0 % NCHUNK]],
                rows_v.at[p0],
                sems[p0],
            ).start()

        def g_body(g, _, bank=bank):
            for phase in range(NBUF):
                t = g * NBUF + phase
                b = t // NCHUNK
                c = lax.rem(t, NCHUNK)
                tn = t + (NBUF - 1)

                @pl.when(tn < NT)
                def _(bank=bank, tn=tn, phase=phase):
                    bn = tn // NCHUNK
                    cn = lax.rem(tn, NCHUNK)
                    pn = (phase + NBUF - 1) % NBUF
                    pltpu.make_async_copy(
                        bank.at[idx_all.at[bn, cn]],
                        rows_v.at[pn],
                        sems[pn],
                    ).start()

                pltpu.make_async_copy(
                    bank.at[idx_all.at[b, c]], rows_v.at[phase], sems[phase]
                ).wait()

                e_chunks = [emb_all[b, pl.ds(16 * j, 16)] for j in range(8)]

                @plsc.parallel_loop(0, 8, unroll=4)
                def kk_body(kk, phase=phase, b=b, c=c, e_chunks=e_chunks):
                    vec = jnp.zeros((16,), jnp.float32)
                    for u in range(16):
                        k = kk * 16 + u
                        p = rows_v[phase, k, pl.ds(0, 16)] * e_chunks[0]
                        for j in range(1, 8):
                            p = p + (rows_v[phase, k, pl.ds(16 * j, 16)]
                                     * e_chunks[j])
                        s = jnp.sum(p)
                        vec = jnp.where(lane == u, s, vec)
                    out_all[pl.ds((b * (KP // 16) + c * 8 + kk) * 16, 16)] = vec
            return ()

        lax.fori_loop(0, NT // NBUF, g_body, ())
        pltpu.sync_copy(out_all, out.at[side, pl.ds(base * KP, B_PER_W * KP)])


def _sc_scores(memory_t, memory_s, inds3, emb):
    mesh = plsc.VectorSubcoreMesh(
        core_axis_name="c", subcore_axis_name="s", num_cores=NC, num_subcores=NS
    )
    return pl.kernel(
        _sc_body,
        out_type=jax.ShapeDtypeStruct((2, BSZ * KP), jnp.float32),
        mesh=mesh,
        compiler_params=pltpu.CompilerParams(
            needs_layout_passes=False, use_tc_tiling_on_sc=False
        ),
        scratch_types=[
            pltpu.VMEM((B_PER_W, NCHUNK, CHUNK), jnp.int32),
            pltpu.VMEM((B_PER_W, FEAT), jnp.float32),
            pltpu.VMEM((NBUF, CHUNK, FEAT), jnp.float32),
            pltpu.VMEM((B_PER_W * KP,), jnp.float32),
            pltpu.SemaphoreType.DMA,
            pltpu.SemaphoreType.DMA,
            pltpu.SemaphoreType.DMA,
            pltpu.SemaphoreType.DMA,
        ],
    )(memory_t, memory_s, inds3, emb)


def _loss_body(ex_ref, out_ref):
    c = jnp.float32(K * (1.0 / N_DATA))
    total = jnp.float32(0.0)
    for side in range(2):
        ex = jnp.exp(ex_ref[side] * (1.0 / T_NCE))
        z = jnp.sum(ex) * (N_DATA / (BSZ * KP))
        pos = ex[:, 0:1] / z
        t1 = jnp.sum(jnp.log(pos / (pos + c + EPS)))
        all_neg = jnp.sum(jnp.log(c / (ex / z + c + EPS)))
        t2 = all_neg - jnp.sum(jnp.log(c / (pos + c + EPS)))
        total = total + (-(t1 + t2) / BSZ)
    out_ref[...] = jnp.broadcast_to(total, (1, 1))


def _loss(ex):
    return pl.pallas_call(
        _loss_body,
        out_shape=jax.ShapeDtypeStruct((1, 1), jnp.float32),
    )(ex)


def kernel(epoch, f_s, f_t, idx, contrast_idx, W_s, b_s, W_t, b_t,
           memory_s, memory_t):
    emb = _embed(f_s, W_s, b_s, f_t, W_t, b_t)
    inds = jnp.concatenate([idx[:, None], contrast_idx], axis=1)
    inds3 = inds.reshape(BSZ, NCHUNK, CHUNK).astype(jnp.int32)
    sc_out = _sc_scores(memory_t, memory_s, inds3, emb)
    return _loss(sc_out.reshape(2, BSZ, KP)).reshape(())


# speedup vs baseline: 1.5058x; 1.5058x over previous
import functools

import jax
import jax.numpy as jnp
from jax import lax
from jax.experimental import pallas as pl
from jax.experimental.pallas import tpu as pltpu
from jax.experimental.pallas import tpu_sc as plsc

EPS = 1e-07
N_DATA = 100000
FEAT = 128
BSZ = 1024
K = 511
P = 1
KP = K + P
T_NCE = 0.07

NC = 2
NS = 16
NW = NC * NS
B_PER_W = BSZ // NW
CHUNK = 128
NCHUNK = KP // CHUNK
NBUF = 2


def _emb_body(fs_ref, ws_ref, bs_ref, ft_ref, wt_ref, bt_ref, out_ref):
    for side, (f, w, b) in enumerate(
        ((fs_ref, ws_ref, bs_ref), (ft_ref, wt_ref, bt_ref))
    ):
        e = jnp.dot(
            f[...], w[...], preferred_element_type=jnp.float32
        ) + b[...]
        e = e / jnp.sqrt(jnp.sum(e * e, axis=1, keepdims=True))
        out_ref[side] = e


def _embed(f_s, W_s, b_s, f_t, W_t, b_t):
    blk = 256
    grid = (BSZ // blk,)
    return pl.pallas_call(
        _emb_body,
        grid=grid,
        in_specs=[
            pl.BlockSpec((blk, f_s.shape[1]), lambda i: (i, 0)),
            pl.BlockSpec(W_s.shape, lambda i: (0, 0)),
            pl.BlockSpec((1, FEAT), lambda i: (0, 0)),
            pl.BlockSpec((blk, f_t.shape[1]), lambda i: (i, 0)),
            pl.BlockSpec(W_t.shape, lambda i: (0, 0)),
            pl.BlockSpec((1, FEAT), lambda i: (0, 0)),
        ],
        out_specs=pl.BlockSpec((2, blk, FEAT), lambda i: (0, i, 0)),
        out_shape=jax.ShapeDtypeStruct((2, BSZ, FEAT), jnp.float32),
    )(f_s, W_s, b_s.reshape(1, FEAT), f_t, W_t, b_t.reshape(1, FEAT))


def _sc_body(mem_t, mem_s, inds, emb, out, idx_all, emb_all, rows_v, out_all,
             sem0, sem1, sem2, sem3):
    wid = lax.axis_index("s") * NC + lax.axis_index("c")
    base = wid * B_PER_W
    lane = lax.iota(jnp.int32, 16)
    sems = (sem0, sem1, sem2, sem3)
    NT = B_PER_W * NCHUNK

    pltpu.sync_copy(inds.at[pl.ds(base, B_PER_W)], idx_all)

    for side in range(2):
        bank = mem_t if side == 0 else mem_s
        pltpu.sync_copy(emb.at[side, pl.ds(base, B_PER_W)], emb_all)

        for p0 in range(NBUF - 1):
            pltpu.make_async_copy(
                bank.at[idx_all.at[p0 // NCHUNK, p0 % NCHUNK]],
                rows_v.at[p0],
                sems[p0],
            ).start()

        def g_body(g, _, bank=bank):
            for phase in range(NBUF):
                t = g * NBUF + phase
                b = t // NCHUNK
                c = lax.rem(t, NCHUNK)
                tn = t + (NBUF - 1)

                @pl.when(tn < NT)
                def _(bank=bank, tn=tn, phase=phase):
                    bn = tn // NCHUNK
                    cn = lax.rem(tn, NCHUNK)
                    pn = (phase + NBUF - 1) % NBUF
                    pltpu.make_async_copy(
                        bank.at[idx_all.at[bn, cn]],
                        rows_v.at[pn],
                        sems[pn],
                    ).start()

                pltpu.make_async_copy(
                    bank.at[idx_all.at[b, c]], rows_v.at[phase], sems[phase]
                ).wait()

                e_chunks = [emb_all[b, pl.ds(16 * j, 16)] for j in range(8)]

                def kk_body(kk, _, phase=phase, b=b, c=c, e_chunks=e_chunks):
                    vec = jnp.zeros((16,), jnp.float32)
                    for u in range(16):
                        k = kk * 16 + u
                        p = rows_v[phase, k, pl.ds(0, 16)] * e_chunks[0]
                        for j in range(1, 8):
                            p = p + (rows_v[phase, k, pl.ds(16 * j, 16)]
                                     * e_chunks[j])
                        s = jnp.sum(p)
                        vec = jnp.where(lane == u, s, vec)
                    out_all[pl.ds((b * (KP // 16) + c * 8 + kk) * 16, 16)] = vec
                    return ()

                lax.fori_loop(0, 8, kk_body, (), unroll=4)
            return ()

        lax.fori_loop(0, NT // NBUF, g_body, ())
        pltpu.sync_copy(out_all, out.at[side, pl.ds(base * KP, B_PER_W * KP)])


def _sc_scores(memory_t, memory_s, inds3, emb):
    mesh = plsc.VectorSubcoreMesh(
        core_axis_name="c", subcore_axis_name="s", num_cores=NC, num_subcores=NS
    )
    return pl.kernel(
        _sc_body,
        out_type=jax.ShapeDtypeStruct((2, BSZ * KP), jnp.float32),
        mesh=mesh,
        compiler_params=pltpu.CompilerParams(
            needs_layout_passes=False, use_tc_tiling_on_sc=False
        ),
        scratch_types=[
            pltpu.VMEM((B_PER_W, NCHUNK, CHUNK), jnp.int32),
            pltpu.VMEM((B_PER_W, FEAT), jnp.float32),
            pltpu.VMEM((NBUF, CHUNK, FEAT), jnp.float32),
            pltpu.VMEM((B_PER_W * KP,), jnp.float32),
            pltpu.SemaphoreType.DMA,
            pltpu.SemaphoreType.DMA,
            pltpu.SemaphoreType.DMA,
            pltpu.SemaphoreType.DMA,
        ],
    )(memory_t, memory_s, inds3, emb)


def _loss_body(ex_ref, out_ref):
    c = jnp.float32(K * (1.0 / N_DATA))
    total = jnp.float32(0.0)
    for side in range(2):
        ex = jnp.exp(ex_ref[side] * (1.0 / T_NCE))
        z = jnp.sum(ex) * (N_DATA / (BSZ * KP))
        pos = ex[:, 0:1] / z
        t1 = jnp.sum(jnp.log(pos / (pos + c + EPS)))
        all_neg = jnp.sum(jnp.log(c / (ex / z + c + EPS)))
        t2 = all_neg - jnp.sum(jnp.log(c / (pos + c + EPS)))
        total = total + (-(t1 + t2) / BSZ)
    out_ref[...] = jnp.broadcast_to(total, (1, 1))


def _loss(ex):
    return pl.pallas_call(
        _loss_body,
        out_shape=jax.ShapeDtypeStruct((1, 1), jnp.float32),
    )(ex)


def kernel(epoch, f_s, f_t, idx, contrast_idx, W_s, b_s, W_t, b_t,
           memory_s, memory_t):
    emb = _embed(f_s, W_s, b_s, f_t, W_t, b_t)
    inds = jnp.concatenate([idx[:, None], contrast_idx], axis=1)
    inds3 = inds.reshape(BSZ, NCHUNK, CHUNK).astype(jnp.int32)
    sc_out = _sc_scores(memory_t, memory_s, inds3, emb)
    return _loss(sc_out.reshape(2, BSZ, KP)).reshape(())


# speedup vs baseline: 1.5060x; 1.0001x over previous
import functools

import jax
import jax.numpy as jnp
from jax import lax
from jax.experimental import pallas as pl
from jax.experimental.pallas import tpu as pltpu
from jax.experimental.pallas import tpu_sc as plsc

EPS = 1e-07
N_DATA = 100000
FEAT = 128
BSZ = 1024
K = 511
P = 1
KP = K + P
T_NCE = 0.07

NC = 2
NS = 16
NW = NC * NS
B_PER_W = BSZ // NW
CHUNK = 128
NCHUNK = KP // CHUNK
NBUF = 2


def _emb_body(fs_ref, ws_ref, bs_ref, ft_ref, wt_ref, bt_ref, out_ref):
    for side, (f, w, b) in enumerate(
        ((fs_ref, ws_ref, bs_ref), (ft_ref, wt_ref, bt_ref))
    ):
        e = jnp.dot(
            f[...], w[...], preferred_element_type=jnp.float32
        ) + b[...]
        e = e / jnp.sqrt(jnp.sum(e * e, axis=1, keepdims=True))
        out_ref[side] = e


def _embed(f_s, W_s, b_s, f_t, W_t, b_t):
    blk = 256
    grid = (BSZ // blk,)
    return pl.pallas_call(
        _emb_body,
        grid=grid,
        in_specs=[
            pl.BlockSpec((blk, f_s.shape[1]), lambda i: (i, 0)),
            pl.BlockSpec(W_s.shape, lambda i: (0, 0)),
            pl.BlockSpec((1, FEAT), lambda i: (0, 0)),
            pl.BlockSpec((blk, f_t.shape[1]), lambda i: (i, 0)),
            pl.BlockSpec(W_t.shape, lambda i: (0, 0)),
            pl.BlockSpec((1, FEAT), lambda i: (0, 0)),
        ],
        out_specs=pl.BlockSpec((2, blk, FEAT), lambda i: (0, i, 0)),
        out_shape=jax.ShapeDtypeStruct((2, BSZ, FEAT), jnp.float32),
    )(f_s, W_s, b_s.reshape(1, FEAT), f_t, W_t, b_t.reshape(1, FEAT))


def _sc_body(mem_t, mem_s, inds, emb, out, idx_all, emb_all, rows_v, out_all,
             sem0, sem1, sem2, sem3):
    wid = lax.axis_index("s") * NC + lax.axis_index("c")
    base = wid * B_PER_W
    lane = lax.iota(jnp.int32, 16)
    fifteen = jnp.full((16,), 15, jnp.int32)
    sems = (sem0, sem1, sem2, sem3)
    NT = B_PER_W * NCHUNK

    pltpu.sync_copy(inds.at[pl.ds(base, B_PER_W)], idx_all)

    for side in range(2):
        bank = mem_t if side == 0 else mem_s
        pltpu.sync_copy(emb.at[side, pl.ds(base, B_PER_W)], emb_all)

        for p0 in range(NBUF - 1):
            pltpu.make_async_copy(
                bank.at[idx_all.at[p0 // NCHUNK, p0 % NCHUNK]],
                rows_v.at[p0],
                sems[p0],
            ).start()

        def g_body(g, _, bank=bank):
            for phase in range(NBUF):
                t = g * NBUF + phase
                b = t // NCHUNK
                c = lax.rem(t, NCHUNK)
                tn = t + (NBUF - 1)

                @pl.when(tn < NT)
                def _(bank=bank, tn=tn, phase=phase):
                    bn = tn // NCHUNK
                    cn = lax.rem(tn, NCHUNK)
                    pn = (phase + NBUF - 1) % NBUF
                    pltpu.make_async_copy(
                        bank.at[idx_all.at[bn, cn]],
                        rows_v.at[pn],
                        sems[pn],
                    ).start()

                pltpu.make_async_copy(
                    bank.at[idx_all.at[b, c]], rows_v.at[phase], sems[phase]
                ).wait()

                e_chunks = [emb_all[b, pl.ds(16 * j, 16)] for j in range(8)]

                def kk_body(kk, _, phase=phase, b=b, c=c, e_chunks=e_chunks):
                    vec = jnp.zeros((16,), jnp.float32)
                    for u in range(16):
                        k = kk * 16 + u
                        p = rows_v[phase, k, pl.ds(0, 16)] * e_chunks[0]
                        for j in range(1, 8):
                            p = p + (rows_v[phase, k, pl.ds(16 * j, 16)]
                                     * e_chunks[j])
                        cs = plsc.cumsum(p)
                        tot = lax.gather(
                            cs,
                            fifteen[:, None],
                            lax.GatherDimensionNumbers(
                                offset_dims=(),
                                collapsed_slice_dims=(0,),
                                start_index_map=(0,),
                            ),
                            (1,),
                            mode=lax.GatherScatterMode.PROMISE_IN_BOUNDS,
                        )
                        vec = jnp.where(lane == u, tot, vec)
                    out_all[pl.ds((b * (KP // 16) + c * 8 + kk) * 16, 16)] = vec
                    return ()

                lax.fori_loop(0, 8, kk_body, (), unroll=4)
            return ()

        lax.fori_loop(0, NT // NBUF, g_body, ())
        pltpu.sync_copy(out_all, out.at[side, pl.ds(base * KP, B_PER_W * KP)])


def _sc_scores(memory_t, memory_s, inds3, emb):
    mesh = plsc.VectorSubcoreMesh(
        core_axis_name="c", subcore_axis_name="s", num_cores=NC, num_subcores=NS
    )
    return pl.kernel(
        _sc_body,
        out_type=jax.ShapeDtypeStruct((2, BSZ * KP), jnp.float32),
        mesh=mesh,
        compiler_params=pltpu.CompilerParams(
            needs_layout_passes=False, use_tc_tiling_on_sc=False
        ),
        scratch_types=[
            pltpu.VMEM((B_PER_W, NCHUNK, CHUNK), jnp.int32),
            pltpu.VMEM((B_PER_W, FEAT), jnp.float32),
            pltpu.VMEM((NBUF, CHUNK, FEAT), jnp.float32),
            pltpu.VMEM((B_PER_W * KP,), jnp.float32),
            pltpu.SemaphoreType.DMA,
            pltpu.SemaphoreType.DMA,
            pltpu.SemaphoreType.DMA,
            pltpu.SemaphoreType.DMA,
        ],
    )(memory_t, memory_s, inds3, emb)


def _loss_body(ex_ref, out_ref):
    c = jnp.float32(K * (1.0 / N_DATA))
    total = jnp.float32(0.0)
    for side in range(2):
        ex = jnp.exp(ex_ref[side] * (1.0 / T_NCE))
        z = jnp.sum(ex) * (N_DATA / (BSZ * KP))
        pos = ex[:, 0:1] / z
        t1 = jnp.sum(jnp.log(pos / (pos + c + EPS)))
        all_neg = jnp.sum(jnp.log(c / (ex / z + c + EPS)))
        t2 = all_neg - jnp.sum(jnp.log(c / (pos + c + EPS)))
        total = total + (-(t1 + t2) / BSZ)
    out_ref[...] = jnp.broadcast_to(total, (1, 1))


def _loss(ex):
    return pl.pallas_call(
        _loss_body,
        out_shape=jax.ShapeDtypeStruct((1, 1), jnp.float32),
    )(ex)


def kernel(epoch, f_s, f_t, idx, contrast_idx, W_s, b_s, W_t, b_t,
           memory_s, memory_t):
    emb = _embed(f_s, W_s, b_s, f_t, W_t, b_t)
    inds = jnp.concatenate([idx[:, None], contrast_idx], axis=1)
    inds3 = inds.reshape(BSZ, NCHUNK, CHUNK).astype(jnp.int32)
    sc_out = _sc_scores(memory_t, memory_s, inds3, emb)
    return _loss(sc_out.reshape(2, BSZ, KP)).reshape(())
